# baseline (speedup 1.0000x reference)
import jax
import jax.numpy as jnp
from jax import lax
from jax.experimental import pallas as pl
from jax.experimental.pallas import tpu as pltpu
from jax.experimental.pallas import tpu_sc as plsc

_N = 10000
_E = 320000
_D = 128
_DLAT = 64
_CH = 80
_NCHUNKS = _E // _CH
_CPW = _NCHUNKS // 32
_CPAD = 4008
_IDXROWS = 136
_NC = 2
_NS = 16
_NW = _NC * _NS
_RPT = 624
_TAIL = _N - _RPT * _NS
_SLOPE = 0.01

_MESH = plsc.VectorSubcoreMesh(core_axis_name="c", subcore_axis_name="s")


def _zero16():
    return jnp.zeros((16,), jnp.float32)


def _sliced(body_per_slice):
    def run(sid):
        row0 = pl.multiple_of(sid * _RPT, 8)
        for p in range(8):
            ln = _CH if p < 7 else _RPT - 7 * _CH
            body_per_slice(row0 + p * _CH, ln)

        @pl.when(sid == _NS - 1)
        def _tail():
            body_per_slice(_RPT * _NS, _TAIL)

    return run


def _worker_range(wid):
    cstart = wid * _CPW
    sa = pl.multiple_of((cstart // 8) * 8, 8)
    delta = cstart - sa
    return cstart, sa, delta


def _fill_buf(ref, value):
    nrow, ncol = ref.shape[-2], ref.shape[-1]
    vec = _zero16() + value

    def zrow(r, carry):
        for c8 in range(ncol // 16):
            if len(ref.shape) == 3:
                ref[0, r, pl.ds(c8 * 16, 16)] = vec
            else:
                ref[r, pl.ds(c8 * 16, 16)] = vec
        return carry

    lax.fori_loop(0, nrow, zrow, 0)


def _agg_kernel_body(x_hbm, src_hbm, dst_hbm, acc_out,
                     sring, idx_d, rows, acc_sh,
                     dsem, isem, gsem0, gsem1, gsem2, ssem0, ssem1, ssem2):
    cid = lax.axis_index("c")
    sid = lax.axis_index("s")
    wid = cid * _NS + sid
    cstart, sa, delta = _worker_range(wid)

    pltpu.async_copy(dst_hbm.at[pl.ds(sa, _IDXROWS)], idx_d, dsem)

    def src_slice(k):
        return src_hbm.at[pl.ds(pl.multiple_of((cstart + k) * _CH, _CH), _CH)]

    pltpu.sync_copy(src_slice(0), sring.at[0])
    pltpu.async_copy(src_slice(1), sring.at[1], isem)

    _fill_buf(rows, 0.0)
    _sliced(lambda off, ln: pltpu.sync_copy(
        rows.at[0, pl.ds(0, ln)], acc_sh.at[pl.ds(off, ln)]))(sid)

    pltpu.make_async_copy(dst_hbm.at[pl.ds(sa, _IDXROWS)], idx_d, dsem).wait()
    plsc.subcore_barrier()

    gsems = (gsem0, gsem1, gsem2)
    ssems = (ssem0, ssem1, ssem2)

    def fire_gather(k, slot):
        pltpu.async_copy(x_hbm.at[sring.at[k % 4]], rows.at[slot],
                         gsems[slot])

    def drain_gather(k, slot):
        pltpu.make_async_copy(x_hbm.at[sring.at[k % 4]], rows.at[slot],
                              gsems[slot]).wait()

    def fire_scat(k, slot):
        pltpu.async_copy(rows.at[slot], acc_sh.at[idx_d.at[delta + k]],
                         ssems[slot], add=True)

    def drain_scat_k(k):
        for slot in (0, 1, 2):
            @pl.when((k % 3) == slot)
            def _d(slot=slot):
                pltpu.make_async_copy(
                    rows.at[slot], acc_sh.at[idx_d.at[delta + k]],
                    ssems[slot]).wait()

    fire_gather(0, 0)

    def body(k, carry):
        nk = k + 1

        @pl.when(k >= 2)
        def _free_rows():
            drain_scat_k(k - 2)

        for slot in (0, 1, 2):
            @pl.when(jnp.logical_and(nk < _CPW, (nk % 3) == slot))
            def _g(slot=slot):
                pltpu.make_async_copy(src_slice(nk), sring.at[nk % 4],
                                      isem).wait()
                fire_gather(nk, slot)

        @pl.when(nk + 1 < _CPW)
        def _prefetch():
            pltpu.async_copy(src_slice(nk + 1), sring.at[(nk + 1) % 4], isem)

        for slot in (0, 1, 2):
            @pl.when((k % 3) == slot)
            def _s(slot=slot):
                drain_gather(k, slot)
                fire_scat(k, slot)

        return carry

    lax.fori_loop(0, _CPW, body, 0)
    drain_scat_k(_CPW - 2)
    drain_scat_k(_CPW - 1)
    plsc.subcore_barrier()

    def wb(off, ln):
        pltpu.sync_copy(acc_sh.at[pl.ds(off, ln)], rows.at[0, pl.ds(0, ln)])
        pltpu.sync_copy(rows.at[0, pl.ds(0, ln)],
                        acc_out.at[cid, pl.ds(off, ln)])

    _sliced(wb)(sid)


_agg = pl.kernel(
    _agg_kernel_body,
    out_type=jax.ShapeDtypeStruct((_NC, _N, _D), jnp.float32),
    mesh=_MESH,
    scratch_types=[
        pltpu.VMEM((4, _CH), jnp.int32),
        pltpu.VMEM((_IDXROWS, _CH), jnp.int32),
        pltpu.VMEM((3, _CH, _D), jnp.float32),
        pltpu.VMEM_SHARED((_N, _D), jnp.float32),
        pltpu.SemaphoreType.DMA,
        pltpu.SemaphoreType.DMA,
        pltpu.SemaphoreType.DMA,
        pltpu.SemaphoreType.DMA,
        pltpu.SemaphoreType.DMA,
        pltpu.SemaphoreType.DMA,
        pltpu.SemaphoreType.DMA,
        pltpu.SemaphoreType.DMA,
    ],
)


def _count_kernel_body(dst_hbm, deg_out, idx_d, rows, deg_sh, isem, ssem):
    cid = lax.axis_index("c")
    sid = lax.axis_index("s")
    wid = cid * _NS + sid
    _, sa, delta = _worker_range(wid)
    ncw = _CPW

    pltpu.async_copy(dst_hbm.at[pl.ds(sa, _IDXROWS)], idx_d, isem)

    _fill_buf(rows, 0.0)
    _sliced(lambda off, ln: pltpu.sync_copy(
        rows.at[0, pl.ds(0, ln)], deg_sh.at[pl.ds(off, ln)]))(sid)
    _fill_buf(rows, 1.0)

    pltpu.make_async_copy(dst_hbm.at[pl.ds(sa, _IDXROWS)], idx_d, isem).wait()
    plsc.subcore_barrier()

    _W = 8

    def fire(k):
        pltpu.async_copy(rows.at[0], deg_sh.at[idx_d.at[delta + k]],
                         ssem, add=True)

    def drain(k):
        pltpu.make_async_copy(rows.at[0], deg_sh.at[idx_d.at[delta + k]],
                              ssem).wait()

    def chunk(k, carry):
        @pl.when(k >= _W)
        def _d():
            drain(k - _W)

        fire(k)
        return carry

    lax.fori_loop(0, ncw, chunk, 0)

    def dr(k, carry):
        drain(k)
        return carry

    lax.fori_loop(jnp.maximum(ncw - _W, 0), ncw, dr, 0)
    plsc.subcore_barrier()

    def wb(off, ln):
        pltpu.sync_copy(deg_sh.at[pl.ds(off, ln)], rows.at[0, pl.ds(0, ln)])
        pltpu.sync_copy(rows.at[0, pl.ds(0, ln)],
                        deg_out.at[cid, pl.ds(off, ln)])

    _sliced(wb)(sid)


_deg_count = pl.kernel(
    _count_kernel_body,
    out_type=jax.ShapeDtypeStruct((_NC, _N, _D), jnp.float32),
    mesh=_MESH,
    scratch_types=[
        pltpu.VMEM((_IDXROWS, _CH), jnp.int32),
        pltpu.VMEM((1, _CH, _D), jnp.float32),
        pltpu.VMEM_SHARED((_N, _D), jnp.float32),
        pltpu.SemaphoreType.DMA,
        pltpu.SemaphoreType.DMA,
    ],
)


def _mean(acc_ref, deg_ref):
    acc = acc_ref[0] + acc_ref[1]
    deg = deg_ref[0, :, 0:1] + deg_ref[1, :, 0:1]
    return acc / jnp.maximum(deg, 1.0)


def _dot(a, b):
    return jnp.dot(a, b, preferred_element_type=jnp.float32,
                   precision=lax.Precision.HIGHEST)


def _layer_body(acc_ref, deg_ref, x_ref, wl_ref, bl_ref, wr_ref, o_ref):
    mean = _mean(acc_ref, deg_ref)
    h = _dot(mean, wl_ref[...]) + bl_ref[...] + _dot(x_ref[...], wr_ref[...])
    o_ref[...] = jnp.where(h >= 0, h, _SLOPE * h)


def _heads_body(acc_ref, deg_ref, h_ref, wml_ref, bml_ref, wmr_ref,
                wvl_ref, bvl_ref, wvr_ref, mu_ref, lv_ref):
    mean = _mean(acc_ref, deg_ref)
    h = h_ref[...]
    mu_ref[...] = _dot(mean, wml_ref[...]) + bml_ref[...] + _dot(h, wmr_ref[...])
    lv_ref[...] = _dot(mean, wvl_ref[...]) + bvl_ref[...] + _dot(h, wvr_ref[...])


_R = 1000
_GRID = _N // _R

_acc_spec = pl.BlockSpec((_NC, _R, _D), lambda i: (0, i, 0))
_deg_spec = _acc_spec
_row_spec = pl.BlockSpec((_R, _D), lambda i: (i, 0))
_w_spec = pl.BlockSpec((_D, _D), lambda i: (0, 0))
_b_spec = pl.BlockSpec((1, _D), lambda i: (0, 0))
_wlat_spec = pl.BlockSpec((_D, _DLAT), lambda i: (0, 0))
_blat_spec = pl.BlockSpec((1, _DLAT), lambda i: (0, 0))
_olat_spec = pl.BlockSpec((_R, _DLAT), lambda i: (i, 0))

_tc_layer = pl.pallas_call(
    _layer_body,
    grid=(_GRID,),
    in_specs=[_acc_spec, _deg_spec, _row_spec, _w_spec, _b_spec, _w_spec],
    out_specs=_row_spec,
    out_shape=jax.ShapeDtypeStruct((_N, _D), jnp.float32),
)

_tc_heads = pl.pallas_call(
    _heads_body,
    grid=(_GRID,),
    in_specs=[_acc_spec, _deg_spec, _row_spec, _wlat_spec, _blat_spec,
              _wlat_spec, _wlat_spec, _blat_spec, _wlat_spec],
    out_specs=(_olat_spec, _olat_spec),
    out_shape=(jax.ShapeDtypeStruct((_N, _DLAT), jnp.float32),
               jax.ShapeDtypeStruct((_N, _DLAT), jnp.float32)),
)


def kernel(x, edge_index, W0l, b0l, W0r, W1l, b1l, W1r,
           Wml, bml, Wmr, Wvl, bvl, Wvr):
    src1d = jnp.pad(edge_index[0], (0, _CPAD * _CH - _E))
    dst2d = jnp.pad(edge_index[1].reshape(_NCHUNKS, _CH),
                    ((0, _CPAD - _NCHUNKS), (0, 0)))

    deg = _deg_count(dst2d)
    acc0 = _agg(x, src1d, dst2d)
    h1 = _tc_layer(acc0, deg, x, W0l, b0l.reshape(1, _D), W0r)
    acc1 = _agg(h1, src1d, dst2d)
    h2 = _tc_layer(acc1, deg, h1, W1l, b1l.reshape(1, _D), W1r)
    acc2 = _agg(h2, src1d, dst2d)
    mu, logvar = _tc_heads(acc2, deg, h2, Wml, bml.reshape(1, _DLAT), Wmr,
                           Wvl, bvl.reshape(1, _DLAT), Wvr)
    return (mu, logvar)


# speedup vs baseline: 1.1231x; 1.1231x over previous
import jax
import jax.numpy as jnp
from jax import lax
from jax.experimental import pallas as pl
from jax.experimental.pallas import tpu as pltpu
from jax.experimental.pallas import tpu_sc as plsc

_N = 10000
_E = 320000
_D = 128
_DLAT = 64
_CH = 80
_NCHUNKS = _E // _CH
_CPW = _NCHUNKS // 32
_CPAD = 4008
_IDXROWS = 136
_NC = 2
_NS = 16
_NW = _NC * _NS
_RPT = 624
_TAIL = _N - _RPT * _NS
_SLOPE = 0.01

_MESH = plsc.VectorSubcoreMesh(core_axis_name="c", subcore_axis_name="s")


def _zero16():
    return jnp.zeros((16,), jnp.float32)


def _sliced(body_per_slice):
    def run(sid):
        row0 = pl.multiple_of(sid * _RPT, 8)
        for p in range(8):
            ln = _CH if p < 7 else _RPT - 7 * _CH
            body_per_slice(row0 + p * _CH, ln)

        @pl.when(sid == _NS - 1)
        def _tail():
            body_per_slice(_RPT * _NS, _TAIL)

    return run


def _worker_range(wid):
    cstart = wid * _CPW
    sa = pl.multiple_of((cstart // 8) * 8, 8)
    delta = cstart - sa
    return cstart, sa, delta


def _fill_buf(ref, value):
    nrow, ncol = ref.shape[-2], ref.shape[-1]
    vec = _zero16() + value

    def zrow(r, carry):
        for c8 in range(ncol // 16):
            if len(ref.shape) == 3:
                ref[0, r, pl.ds(c8 * 16, 16)] = vec
            else:
                ref[r, pl.ds(c8 * 16, 16)] = vec
        return carry

    lax.fori_loop(0, nrow, zrow, 0)


def _agg_kernel_body(x_hbm, src_hbm, dst_hbm, acc_out,
                     sring, idx_d, rows, acc_sh,
                     dsem, isem, gsem0, gsem1, gsem2, ssem0, ssem1, ssem2):
    cid = lax.axis_index("c")
    sid = lax.axis_index("s")
    wid = cid * _NS + sid
    cstart, sa, delta = _worker_range(wid)

    pltpu.async_copy(dst_hbm.at[pl.ds(sa, _IDXROWS)], idx_d, dsem)

    def src_slice(k):
        return src_hbm.at[pl.ds(pl.multiple_of((cstart + k) * _CH, _CH), _CH)]

    pltpu.sync_copy(src_slice(0), sring.at[0])
    pltpu.async_copy(src_slice(1), sring.at[1], isem)

    _fill_buf(rows, 0.0)
    _sliced(lambda off, ln: pltpu.sync_copy(
        rows.at[0, pl.ds(0, ln)], acc_sh.at[pl.ds(off, ln)]))(sid)

    pltpu.make_async_copy(dst_hbm.at[pl.ds(sa, _IDXROWS)], idx_d, dsem).wait()
    plsc.subcore_barrier()

    gsems = (gsem0, gsem1, gsem2)
    ssems = (ssem0, ssem1, ssem2)

    def fire_gather(k, slot):
        pltpu.async_copy(x_hbm.at[sring.at[k % 4]], rows.at[slot],
                         gsems[slot])

    def drain_gather(k, slot):
        pltpu.make_async_copy(x_hbm.at[sring.at[k % 4]], rows.at[slot],
                              gsems[slot]).wait()

    def fire_scat(k, slot):
        pltpu.async_copy(rows.at[slot], acc_sh.at[idx_d.at[delta + k]],
                         ssems[slot], add=True)

    def drain_scat_k(k):
        for slot in (0, 1, 2):
            @pl.when((k % 3) == slot)
            def _d(slot=slot):
                pltpu.make_async_copy(
                    rows.at[slot], acc_sh.at[idx_d.at[delta + k]],
                    ssems[slot]).wait()

    fire_gather(0, 0)

    def body(k, carry):
        nk = k + 1

        @pl.when(k >= 2)
        def _free_rows():
            drain_scat_k(k - 2)

        for slot in (0, 1, 2):
            @pl.when(jnp.logical_and(nk < _CPW, (nk % 3) == slot))
            def _g(slot=slot):
                pltpu.make_async_copy(src_slice(nk), sring.at[nk % 4],
                                      isem).wait()
                fire_gather(nk, slot)

        @pl.when(nk + 1 < _CPW)
        def _prefetch():
            pltpu.async_copy(src_slice(nk + 1), sring.at[(nk + 1) % 4], isem)

        for slot in (0, 1, 2):
            @pl.when((k % 3) == slot)
            def _s(slot=slot):
                drain_gather(k, slot)
                fire_scat(k, slot)

        return carry

    lax.fori_loop(0, _CPW, body, 0)
    drain_scat_k(_CPW - 2)
    drain_scat_k(_CPW - 1)
    plsc.subcore_barrier()

    def wb(off, ln):
        pltpu.sync_copy(acc_sh.at[pl.ds(off, ln)], rows.at[0, pl.ds(0, ln)])
        pltpu.sync_copy(rows.at[0, pl.ds(0, ln)],
                        acc_out.at[cid, pl.ds(off, ln)])

    _sliced(wb)(sid)


_agg = pl.kernel(
    _agg_kernel_body,
    out_type=jax.ShapeDtypeStruct((_NC, _N, _D), jnp.float32),
    mesh=_MESH,
    scratch_types=[
        pltpu.VMEM((4, _CH), jnp.int32),
        pltpu.VMEM((_IDXROWS, _CH), jnp.int32),
        pltpu.VMEM((3, _CH, _D), jnp.float32),
        pltpu.VMEM_SHARED((_N, _D), jnp.float32),
        pltpu.SemaphoreType.DMA,
        pltpu.SemaphoreType.DMA,
        pltpu.SemaphoreType.DMA,
        pltpu.SemaphoreType.DMA,
        pltpu.SemaphoreType.DMA,
        pltpu.SemaphoreType.DMA,
        pltpu.SemaphoreType.DMA,
        pltpu.SemaphoreType.DMA,
    ],
)


_DW = 16


def _count_kernel_body(dst_hbm, deg_out, idx_d, rows, deg_sh, isem, ssem):
    cid = lax.axis_index("c")
    sid = lax.axis_index("s")
    wid = cid * _NS + sid
    _, sa, delta = _worker_range(wid)
    ncw = _CPW

    pltpu.async_copy(dst_hbm.at[pl.ds(sa, _IDXROWS)], idx_d, isem)

    _fill_buf(rows, 0.0)
    _sliced(lambda off, ln: pltpu.sync_copy(
        rows.at[0, pl.ds(0, ln)], deg_sh.at[pl.ds(off, ln)]))(sid)
    _fill_buf(rows, 1.0)

    pltpu.make_async_copy(dst_hbm.at[pl.ds(sa, _IDXROWS)], idx_d, isem).wait()
    plsc.subcore_barrier()

    _W = 8

    def fire(k):
        pltpu.async_copy(rows.at[0], deg_sh.at[idx_d.at[delta + k]],
                         ssem, add=True)

    def drain(k):
        pltpu.make_async_copy(rows.at[0], deg_sh.at[idx_d.at[delta + k]],
                              ssem).wait()

    def chunk(k, carry):
        @pl.when(k >= _W)
        def _d():
            drain(k - _W)

        fire(k)
        return carry

    lax.fori_loop(0, ncw, chunk, 0)

    def dr(k, carry):
        drain(k)
        return carry

    lax.fori_loop(jnp.maximum(ncw - _W, 0), ncw, dr, 0)
    plsc.subcore_barrier()

    def wb(off, ln):
        pltpu.sync_copy(deg_sh.at[pl.ds(off, ln)], rows.at[0, pl.ds(0, ln)])
        pltpu.sync_copy(rows.at[0, pl.ds(0, ln)],
                        deg_out.at[cid, pl.ds(off, ln)])

    _sliced(wb)(sid)


_deg_count = pl.kernel(
    _count_kernel_body,
    out_type=jax.ShapeDtypeStruct((_NC, _N, _DW), jnp.float32),
    mesh=_MESH,
    compiler_params=pltpu.CompilerParams(use_tc_tiling_on_sc=False),
    scratch_types=[
        pltpu.VMEM((_IDXROWS, _CH), jnp.int32),
        pltpu.VMEM((1, _CH, _DW), jnp.float32),
        pltpu.VMEM_SHARED((_N, _DW), jnp.float32),
        pltpu.SemaphoreType.DMA,
        pltpu.SemaphoreType.DMA,
    ],
)


def _mean(acc_ref, deg_ref):
    acc = acc_ref[0] + acc_ref[1]
    deg = deg_ref[0, :, 0:1] + deg_ref[1, :, 0:1]
    return acc / jnp.maximum(deg, 1.0)


def _dot(a, b):
    return jnp.dot(a, b, preferred_element_type=jnp.float32,
                   precision=lax.Precision.HIGHEST)


def _layer_body(acc_ref, deg_ref, x_ref, wl_ref, bl_ref, wr_ref, o_ref):
    mean = _mean(acc_ref, deg_ref)
    h = _dot(mean, wl_ref[...]) + bl_ref[...] + _dot(x_ref[...], wr_ref[...])
    o_ref[...] = jnp.where(h >= 0, h, _SLOPE * h)


def _heads_body(acc_ref, deg_ref, h_ref, wml_ref, bml_ref, wmr_ref,
                wvl_ref, bvl_ref, wvr_ref, mu_ref, lv_ref):
    mean = _mean(acc_ref, deg_ref)
    h = h_ref[...]
    mu_ref[...] = _dot(mean, wml_ref[...]) + bml_ref[...] + _dot(h, wmr_ref[...])
    lv_ref[...] = _dot(mean, wvl_ref[...]) + bvl_ref[...] + _dot(h, wvr_ref[...])


_R = 1000
_GRID = _N // _R

_acc_spec = pl.BlockSpec((_NC, _R, _D), lambda i: (0, i, 0))
_deg_spec = pl.BlockSpec((_NC, _R, _DW), lambda i: (0, i, 0))
_row_spec = pl.BlockSpec((_R, _D), lambda i: (i, 0))
_w_spec = pl.BlockSpec((_D, _D), lambda i: (0, 0))
_b_spec = pl.BlockSpec((1, _D), lambda i: (0, 0))
_wlat_spec = pl.BlockSpec((_D, _DLAT), lambda i: (0, 0))
_blat_spec = pl.BlockSpec((1, _DLAT), lambda i: (0, 0))
_olat_spec = pl.BlockSpec((_R, _DLAT), lambda i: (i, 0))

_tc_layer = pl.pallas_call(
    _layer_body,
    grid=(_GRID,),
    in_specs=[_acc_spec, _deg_spec, _row_spec, _w_spec, _b_spec, _w_spec],
    out_specs=_row_spec,
    out_shape=jax.ShapeDtypeStruct((_N, _D), jnp.float32),
)

_tc_heads = pl.pallas_call(
    _heads_body,
    grid=(_GRID,),
    in_specs=[_acc_spec, _deg_spec, _row_spec, _wlat_spec, _blat_spec,
              _wlat_spec, _wlat_spec, _blat_spec, _wlat_spec],
    out_specs=(_olat_spec, _olat_spec),
    out_shape=(jax.ShapeDtypeStruct((_N, _DLAT), jnp.float32),
               jax.ShapeDtypeStruct((_N, _DLAT), jnp.float32)),
)


def kernel(x, edge_index, W0l, b0l, W0r, W1l, b1l, W1r,
           Wml, bml, Wmr, Wvl, bvl, Wvr):
    src1d = jnp.pad(edge_index[0], (0, _CPAD * _CH - _E))
    dst2d = jnp.pad(edge_index[1].reshape(_NCHUNKS, _CH),
                    ((0, _CPAD - _NCHUNKS), (0, 0)))

    deg = _deg_count(dst2d)
    acc0 = _agg(x, src1d, dst2d)
    h1 = _tc_layer(acc0, deg, x, W0l, b0l.reshape(1, _D), W0r)
    acc1 = _agg(h1, src1d, dst2d)
    h2 = _tc_layer(acc1, deg, h1, W1l, b1l.reshape(1, _D), W1r)
    acc2 = _agg(h2, src1d, dst2d)
    mu, logvar = _tc_heads(acc2, deg, h2, Wml, bml.reshape(1, _DLAT), Wmr,
                           Wvl, bvl.reshape(1, _DLAT), Wvr)
    return (mu, logvar)
